# baseline (speedup 1.0000x reference)
import functools

import jax
import jax.numpy as jnp
from jax import lax
from jax.experimental import pallas as pl
from jax.experimental.pallas import tpu as pltpu
from jax.experimental.pallas import tpu_sc as plsc

V = 1000000
D = 64
B = 4096
H = 200
NC, NS = 2, 16
NW = NC * NS
TOTAL = B * H
PER_W = TOTAL // NW
G = 512
NG = PER_W // G

_mesh = plsc.VectorSubcoreMesh(core_axis_name="c", subcore_axis_name="s")


@functools.partial(
    pl.kernel,
    out_type=jax.ShapeDtypeStruct((TOTAL, D), jnp.float32),
    mesh=_mesh,
    scratch_types=[
        pltpu.VMEM((PER_W,), jnp.int32),
        pltpu.VMEM((G, D), jnp.float32),
        pltpu.VMEM((G, D), jnp.float32),
        pltpu.SemaphoreType.DMA,
        pltpu.SemaphoreType.DMA,
        pltpu.SemaphoreType.DMA,
        pltpu.SemaphoreType.DMA,
    ],
    compiler_params=pltpu.CompilerParams(use_tc_tiling_on_sc=False),
)
def _gather_kernel(table_hbm, idx_hbm, out_hbm, idx_v, ra, rb, gsa, gsb, osa, osb):
    wid = lax.axis_index("s") * NC + lax.axis_index("c")
    base = wid * PER_W
    pltpu.sync_copy(idx_hbm.at[pl.ds(base, PER_W)], idx_v)

    def fire_g(group, buf, gsem):
        pltpu.async_copy(table_hbm.at[idx_v.at[pl.ds(group * G, G)]], buf, gsem)

    def drain_g(buf, gsem):
        pltpu.make_async_copy(table_hbm.at[idx_v.at[pl.ds(0, G)]], buf, gsem).wait()

    def fire_w(group, buf, osem):
        pltpu.async_copy(buf, out_hbm.at[pl.ds(base + group * G, G)], osem)

    def drain_w(buf, osem):
        pltpu.make_async_copy(buf, out_hbm.at[pl.ds(base, G)], osem).wait()

    fire_g(0, ra, gsa)
    fire_g(1, rb, gsb)
    drain_g(ra, gsa)
    fire_w(0, ra, osa)

    @pl.loop(0, (NG - 2) // 2)
    def _(t):
        g = 2 * t + 1
        drain_w(ra, osa)
        fire_g(g + 1, ra, gsa)
        drain_g(rb, gsb)
        fire_w(g, rb, osb)
        g2 = g + 1
        drain_w(rb, osb)
        fire_g(g2 + 1, rb, gsb)
        drain_g(ra, gsa)
        fire_w(g2, ra, osa)

    drain_g(rb, gsb)
    fire_w(NG - 1, rb, osb)
    drain_w(ra, osa)
    drain_w(rb, osb)


def kernel(indices, table):
    flat2 = indices.T.reshape(-1).astype(jnp.int32)

    t128 = lax.optimization_barrier(table.reshape(V // 2, 2 * D))
    tlin = t128.reshape(V, D)

    out = _gather_kernel(tlin, flat2)

    return jnp.transpose(out.reshape(H, B, D), (1, 0, 2))
